# Initial kernel scaffold; baseline (speedup 1.0000x reference)
import jax
import jax.numpy as jnp
from jax.experimental import pallas as pl


def kernel(x, edge_index, edge_type, W1, b1, Wr1, br1, Wr2, br2, Wp1, bp1, Wp2, bp2, Wq1, bq1, Wq2, bq2):
    raise NotImplementedError("write your pallas kernel here")


# speedup vs baseline: 3.3544x; 3.3544x over previous
import functools

import jax
import jax.numpy as jnp
from jax import lax
from jax.experimental import pallas as pl
from jax.experimental.pallas import tpu as pltpu
from jax.experimental.pallas import tpu_sc as plsc

N = 100000
E = 1600000
T = 7
C = 32
NSHARD = 8
SHARD = 12544
GARB = SHARD
ACC_ROWS = SHARD + 128
K = 1024
G = K // 128
EPT = E // 16
NCH = (EPT + K - 1) // K
E_PAD = 15 * EPT + NCH * K


def _sc_conv(y, srcp, dstp, typp):
    mesh = plsc.VectorSubcoreMesh(core_axis_name="c", subcore_axis_name="s")

    @functools.partial(
        pl.kernel,
        mesh=mesh,
        out_type=jax.ShapeDtypeStruct((NSHARD * SHARD, 128), jnp.float32),
        scratch_types=[
            pltpu.VMEM((K,), jnp.int32),
            pltpu.VMEM((K,), jnp.int32),
            pltpu.VMEM((K,), jnp.int32),
            pltpu.VMEM((G, 128), jnp.int32),
            pltpu.VMEM((G, 128), jnp.int32),
            pltpu.VMEM((128, 128), jnp.float32),
            pltpu.VMEM_SHARED((ACC_ROWS, 128), jnp.float32),
            pltpu.SemaphoreType.DMA,
        ],
    )
    def k(y_hbm, src_hbm, dst_hbm, typ_hbm, out_hbm,
          src_v, dst_v, typ_v, gidx_v, didx_v, rows_v, acc_sh, sem):
        c = lax.axis_index("c")
        s = lax.axis_index("s")

        base_e = s * EPT

        def one_pass(p, carry):
            lo = (2 * p + c) * SHARD

            def zrow(i, carry2):
                for half in range(8):
                    rows_v[i, pl.ds(half * 16, 16)] = jnp.zeros((16,),
                                                                jnp.float32)
                return carry2
            lax.fori_loop(0, 128, zrow, 0)

            def zacc(i, carry2):
                blk = s + i * 16

                @pl.when(blk < ACC_ROWS // 128)
                def _():
                    pltpu.sync_copy(rows_v, acc_sh.at[pl.ds(blk * 128, 128)])
                return carry2
            lax.fori_loop(0, (ACC_ROWS // 128 + 15) // 16, zacc, 0)
            plsc.subcore_barrier()

            def chunk(ci, carry2):
                off = base_e + ci * K
                pltpu.sync_copy(src_hbm.at[pl.ds(off, K)], src_v)
                pltpu.sync_copy(dst_hbm.at[pl.ds(off, K)], dst_v)
                pltpu.sync_copy(typ_hbm.at[pl.ds(off, K)], typ_v)
                loc0 = ci * K

                def grp(j, carry3):
                    sv = src_v[pl.ds(j * 16, 16)]
                    dv = dst_v[pl.ds(j * 16, 16)]
                    tv = typ_v[pl.ds(j * 16, 16)]
                    pos = loc0 + j * 16 + lax.iota(jnp.int32, 16)
                    u = dv - lo
                    m = (pos < EPT) & (u >= 0) & (u < SHARD)
                    gidx_v[j // 8, pl.ds((j % 8) * 16, 16)] = tv * N + sv
                    didx_v[j // 8, pl.ds((j % 8) * 16, 16)] = jnp.where(m, u, GARB)
                    return carry3
                lax.fori_loop(0, K // 16, grp, 0)

                for t in range(G):
                    pltpu.async_copy(y_hbm.at[gidx_v.at[t]], rows_v,
                                     sem).wait()
                    pltpu.sync_copy(rows_v, acc_sh.at[didx_v.at[t]],
                                    add=True)
                return carry2
            lax.fori_loop(0, NCH, chunk, 0)
            plsc.subcore_barrier()

            r = s * (SHARD // 16)
            pltpu.sync_copy(acc_sh.at[pl.ds(r, SHARD // 16)],
                            out_hbm.at[pl.ds(lo + r, SHARD // 16)])
            plsc.subcore_barrier()
            return carry
        lax.fori_loop(0, NSHARD // 2, one_pass, 0)

    return k(y, srcp, dstp, typp)


def _build_y(h, Wpad):
    NB = 1000
    cin = h.shape[1]

    def body(h_ref, w_ref, y_ref):
        y_ref[...] = jnp.dot(h_ref[...], w_ref[0],
                             preferred_element_type=jnp.float32)

    return pl.pallas_call(
        body,
        grid=(T, N // NB),
        in_specs=[
            pl.BlockSpec((NB, cin), lambda t, i: (i, 0)),
            pl.BlockSpec((1, cin, 128), lambda t, i: (t, 0, 0)),
        ],
        out_specs=pl.BlockSpec((NB, 128), lambda t, i: (t * (N // NB) + i, 0)),
        out_shape=jax.ShapeDtypeStruct((T * N, 128), jnp.float32),
    )(h, Wpad)


def _bias_gelu(a, b):
    NB = 2000

    def body(a_ref, b_ref, o_ref):
        o_ref[...] = jax.nn.gelu(a_ref[...] + b_ref[...])

    return pl.pallas_call(
        body,
        grid=(N // NB,),
        in_specs=[
            pl.BlockSpec((NB, C), lambda i: (i, 0)),
            pl.BlockSpec((1, C), lambda i: (0, 0)),
        ],
        out_specs=pl.BlockSpec((NB, C), lambda i: (i, 0)),
        out_shape=jax.ShapeDtypeStruct((N, C), jnp.float32),
    )(a, b)


def _resid_gelu(h1, c3, br2):
    NB = 2000

    def body(h_ref, c_ref, b_ref, o_ref):
        o_ref[...] = jax.nn.gelu(h_ref[...] + c_ref[...] + b_ref[...])

    return pl.pallas_call(
        body,
        grid=(N // NB,),
        in_specs=[
            pl.BlockSpec((NB, C), lambda i: (i, 0)),
            pl.BlockSpec((NB, C), lambda i: (i, 0)),
            pl.BlockSpec((1, C), lambda i: (0, 0)),
        ],
        out_specs=pl.BlockSpec((NB, C), lambda i: (i, 0)),
        out_shape=jax.ShapeDtypeStruct((N, C), jnp.float32),
    )(h1, c3, br2)


def _heads(h, Wp1, bp1, Wp2, bp2, Wq1, bq1, Wq2, bq2):
    NB = 2000

    def body(h_ref, wp1, bp1r, wp2, bp2r, wq1, bq1r, wq2, bq2r, o_ref):
        hb = h_ref[...]
        g1 = jax.nn.gelu(jnp.dot(hb, wp1[...],
                                 preferred_element_type=jnp.float32) + bp1r[...])
        o1 = jnp.dot(g1, wp2[...], preferred_element_type=jnp.float32) + bp2r[...]
        g2 = jax.nn.gelu(jnp.dot(hb, wq1[...],
                                 preferred_element_type=jnp.float32) + bq1r[...])
        o2 = jnp.dot(g2, wq2[...], preferred_element_type=jnp.float32) + bq2r[...]
        o_ref[...] = jnp.concatenate([o1, o2], axis=1)

    full = lambda shape: pl.BlockSpec(shape, lambda i: tuple(0 for _ in shape))
    return pl.pallas_call(
        body,
        grid=(N // NB,),
        in_specs=[
            pl.BlockSpec((NB, C), lambda i: (i, 0)),
            full((C, 32)), full((1, 32)), full((32, 2)), full((1, 2)),
            full((C, 32)), full((1, 32)), full((32, 4)), full((1, 4)),
        ],
        out_specs=pl.BlockSpec((NB, 6), lambda i: (i, 0)),
        out_shape=jax.ShapeDtypeStruct((N, 6), jnp.float32),
    )(h, Wp1, bp1, Wp2, bp2, Wq1, bq1, Wq2, bq2)


def kernel(x, edge_index, edge_type, W1, b1, Wr1, br1, Wr2, br2,
           Wp1, bp1, Wp2, bp2, Wq1, bq1, Wq2, bq2):
    src = edge_index[0]
    dst = edge_index[1]
    pad = E_PAD - E
    srcp = jnp.pad(src, (0, pad))
    dstp = jnp.pad(dst, (0, pad))
    typp = jnp.pad(edge_type, (0, pad))

    b1r = b1.reshape(1, C)
    br1r = br1.reshape(1, C)
    br2r = br2.reshape(1, C)
    wp = lambda W: jnp.pad(W, ((0, 0), (0, 0), (0, 128 - C)))
    W1p, Wr1p, Wr2p = wp(W1), wp(Wr1), wp(Wr2)

    cut = lambda a: a[:N, :C]
    c1 = cut(_sc_conv(_build_y(x, W1p), srcp, dstp, typp))
    h1 = _bias_gelu(c1, b1r)
    c2 = cut(_sc_conv(_build_y(h1, Wr1p), srcp, dstp, typp))
    r1 = _bias_gelu(c2, br1r)
    c3 = cut(_sc_conv(_build_y(r1, Wr2p), srcp, dstp, typp))
    h = _resid_gelu(h1, c3, br2r)
    return _heads(h, Wp1, bp1.reshape(1, 32), Wp2, bp2.reshape(1, 2),
                  Wq1, bq1.reshape(1, 32), Wq2, bq2.reshape(1, 4))
